# baseline (speedup 1.0000x reference)
import math
import functools

import jax
import jax.numpy as jnp
from jax import lax
from jax.experimental import pallas as pl
from jax.experimental.pallas import tpu as pltpu
from jax.experimental.pallas import tpu_sc as plsc

VOCAB = 1000000
EMB = 32
SCALE = math.sqrt(float(EMB))

NC = 2
NS = 16
NW = NC * NS

SEQ = 200
NTOK = 4096
IW = NTOK // NW
K = 5
NCHUNK = SEQ // K
G = NCHUNK // 2
TPAD = 136

_mesh = plsc.VectorSubcoreMesh(core_axis_name="c", subcore_axis_name="s")


@functools.partial(
    pl.kernel,
    out_type=jax.ShapeDtypeStruct((SEQ, EMB // 8, NW, 8, IW), jnp.float32),
    mesh=_mesh,
    scratch_types=[
        pltpu.VMEM((K, IW), jnp.int32),
        pltpu.VMEM((K, IW), jnp.int32),
        pltpu.VMEM((K * IW, EMB), jnp.float32),
        pltpu.VMEM((K * IW, EMB), jnp.float32),
        pltpu.VMEM((K, EMB // 8, 8, TPAD), jnp.float32),
        pltpu.VMEM((K, EMB // 8, 8, TPAD), jnp.float32),
        pltpu.SemaphoreType.DMA,
        pltpu.SemaphoreType.DMA,
        pltpu.SemaphoreType.DMA,
        pltpu.SemaphoreType.DMA,
    ],
    compiler_params=pltpu.CompilerParams(
        use_tc_tiling_on_sc=False, needs_layout_passes=False
    ),
)
def _emb_kernel(tok_hbm, tab_hbm, out_hbm, idx0, idx1, rows0, rows1,
                tb0, tb1, sg0, sg1, sw0, sw1):
    wid = lax.axis_index("s") * NC + lax.axis_index("c")
    i0 = wid * IW
    idx = (idx0, idx1)
    rows = (rows0, rows1)
    tb = (tb0, tb1)
    sg = (sg0, sg1)
    sw = (sw0, sw1)

    ev = lax.iota(jnp.int32, 16)
    re_v = ev & 7

    def fire(ci, b):
        rb = ci * K
        pltpu.sync_copy(tok_hbm.at[pl.ds(rb, K), pl.ds(i0, IW)], idx[b])
        for j in range(K):
            pltpu.async_copy(
                tab_hbm.at[idx[b].at[j]],
                rows[b].at[pl.ds(j * IW, IW)],
                sg[b],
            )

    def drain_gather(b):
        pltpu.make_async_copy(
            tab_hbm.at[pl.ds(0, K * IW)],
            rows[b],
            sg[b],
        ).wait()

    def transpose_scale(b):
        for kk in range(K):
            kvec = jnp.full((16,), kk, dtype=jnp.int32)

            @plsc.parallel_loop(0, IW, step=1, unroll=4)
            def _(i):
                ivec = jnp.full((16,), i, dtype=jnp.int32)
                x0 = rows[b][kk * IW + i, pl.ds(0, 16)]
                plsc.store_scatter(tb[b], [kvec, ev >> 3, re_v, ivec], x0)
                x1 = rows[b][kk * IW + i, pl.ds(16, 16)]
                plsc.store_scatter(tb[b], [kvec, 2 + (ev >> 3), re_v, ivec], x1)

    def writeback(ci, b):
        rb = ci * K
        pltpu.async_copy(
            tb[b].at[:, :, :, pl.ds(0, IW)],
            out_hbm.at[pl.ds(rb, K), :, wid, :, :],
            sw[b],
        )

    def drain_wb(b):
        pltpu.make_async_copy(
            tb[b].at[:, :, :, pl.ds(0, IW)],
            out_hbm.at[pl.ds(0, K), :, 0, :, :],
            sw[b],
        ).wait()

    fire(0, 0)

    def outer(g, carry):
        ci0 = 2 * g

        @pl.when(g > 0)
        def _():
            drain_wb(1)

        fire(ci0 + 1, 1)
        drain_gather(0)
        transpose_scale(0)
        writeback(ci0, 0)

        drain_gather(1)
        transpose_scale(1)

        @pl.when(g < G - 1)
        def _():
            drain_wb(0)
            fire(ci0 + 2, 0)

        writeback(ci0 + 1, 1)
        return carry

    lax.fori_loop(0, G, outer, 0)
    drain_wb(0)
    drain_wb(1)


W = 2048
_NBLK = -(-VOCAB // (4 * W))
PROWS = _NBLK * W
_LAST_BLK = (VOCAB - 1) // W


def _tab_transpose_body(t0_ref, t1_ref, t2_ref, t3_ref, out_ref):
    ident = jnp.eye(EMB, dtype=jnp.float32) * SCALE
    for r, ref in enumerate((t0_ref, t1_ref, t2_ref, t3_ref)):
        x = ref[...]
        y = lax.dot_general(x, ident, (((0,), (0,)), ((), ())),
                            preferred_element_type=jnp.float32)
        out_ref[:, r * EMB:(r + 1) * EMB] = y


def _transpose_table(table_t):
    def _imap(r, g):
        return (0, jnp.minimum(4 * g + r, _LAST_BLK))

    specs = [
        pl.BlockSpec((EMB, W), functools.partial(_imap, r)) for r in range(4)
    ]
    return pl.pallas_call(
        _tab_transpose_body,
        grid=(_NBLK,),
        in_specs=specs,
        out_specs=pl.BlockSpec((W, 4 * EMB), lambda g: (g, 0)),
        out_shape=jax.ShapeDtypeStruct((PROWS, 4 * EMB), jnp.float32),
    )(table_t, table_t, table_t, table_t)


def kernel(tokens, table):
    v = tokens.T.astype(jnp.int32)
    tok_t = 4 * ((v >> 13) * W + (v & (W - 1))) + ((v >> 11) & 3)
    tab_rm = _transpose_table(table.T)
    tab2 = tab_rm.reshape(4 * PROWS, EMB)
    out5 = _emb_kernel(tok_t, tab2)
    out = jnp.transpose(out5, (2, 4, 0, 1, 3))
    return out.reshape(NTOK, SEQ, EMB)


# speedup vs baseline: 1.0342x; 1.0342x over previous
import math
import functools

import jax
import jax.numpy as jnp
from jax import lax
from jax.experimental import pallas as pl
from jax.experimental.pallas import tpu as pltpu
from jax.experimental.pallas import tpu_sc as plsc

VOCAB = 1000000
EMB = 32
SCALE = math.sqrt(float(EMB))

NC = 2
NS = 16
NW = NC * NS

SEQ = 200
NTOK = 4096
IW = NTOK // NW
K = 5
NCHUNK = SEQ // K
G = NCHUNK // 2
TPAD = 136

_mesh = plsc.VectorSubcoreMesh(core_axis_name="c", subcore_axis_name="s")


@functools.partial(
    pl.kernel,
    out_type=jax.ShapeDtypeStruct((SEQ, EMB // 8, NW, 8, IW), jnp.float32),
    mesh=_mesh,
    scratch_types=[
        pltpu.VMEM((K, IW), jnp.int32),
        pltpu.VMEM((K, IW), jnp.int32),
        pltpu.VMEM((K * IW, EMB), jnp.float32),
        pltpu.VMEM((K * IW, EMB), jnp.float32),
        pltpu.VMEM((K, EMB // 8, 8, TPAD), jnp.float32),
        pltpu.VMEM((K, EMB // 8, 8, TPAD), jnp.float32),
        pltpu.SemaphoreType.DMA,
        pltpu.SemaphoreType.DMA,
        pltpu.SemaphoreType.DMA,
        pltpu.SemaphoreType.DMA,
    ],
    compiler_params=pltpu.CompilerParams(
        use_tc_tiling_on_sc=False, needs_layout_passes=False
    ),
)
def _emb_kernel(tok_hbm, tab_hbm, out_hbm, idx0, idx1, rows0, rows1,
                tb0, tb1, sg0, sg1, sw0, sw1):
    wid = lax.axis_index("s") * NC + lax.axis_index("c")
    i0 = wid * IW
    idx = (idx0, idx1)
    rows = (rows0, rows1)
    tb = (tb0, tb1)
    sg = (sg0, sg1)
    sw = (sw0, sw1)

    ev = lax.iota(jnp.int32, 16)
    re_v = ev & 7

    def fire(ci, b):
        rb = ci * K
        pltpu.sync_copy(tok_hbm.at[pl.ds(rb, K), pl.ds(i0, IW)], idx[b])
        for j in range(K):
            pltpu.async_copy(
                tab_hbm.at[idx[b].at[j]],
                rows[b].at[pl.ds(j * IW, IW)],
                sg[b],
            )

    def drain_gather(b):
        pltpu.make_async_copy(
            tab_hbm.at[pl.ds(0, K * IW)],
            rows[b],
            sg[b],
        ).wait()

    def transpose_scale(b):
        for kk in range(K):
            kvec = jnp.full((16,), kk, dtype=jnp.int32)

            @plsc.parallel_loop(0, IW, step=1, unroll=4)
            def _(i):
                ivec = jnp.full((16,), i, dtype=jnp.int32)
                x0 = rows[b][kk * IW + i, pl.ds(0, 16)]
                plsc.store_scatter(tb[b], [kvec, ev >> 3, re_v, ivec], x0)
                x1 = rows[b][kk * IW + i, pl.ds(16, 16)]
                plsc.store_scatter(tb[b], [kvec, 2 + (ev >> 3), re_v, ivec], x1)

    def writeback(ci, b):
        rb = ci * K
        pltpu.async_copy(
            tb[b].at[:, :, :, pl.ds(0, IW)],
            out_hbm.at[pl.ds(rb, K), :, wid, :, :],
            sw[b],
        )

    def drain_wb(b):
        pltpu.make_async_copy(
            tb[b].at[:, :, :, pl.ds(0, IW)],
            out_hbm.at[pl.ds(0, K), :, 0, :, :],
            sw[b],
        ).wait()

    fire(0, 0)

    def outer(g, carry):
        ci0 = 2 * g

        @pl.when(g > 0)
        def _():
            drain_wb(1)

        fire(ci0 + 1, 1)
        drain_gather(0)
        transpose_scale(0)
        writeback(ci0, 0)

        drain_gather(1)
        transpose_scale(1)

        @pl.when(g < G - 1)
        def _():
            drain_wb(0)
            fire(ci0 + 2, 0)

        writeback(ci0 + 1, 1)
        return carry

    lax.fori_loop(0, G, outer, 0)
    drain_wb(0)
    drain_wb(1)


W = 8192
_NBLK = -(-VOCAB // (4 * W))
PROWS = _NBLK * W
_LAST_BLK = (VOCAB - 1) // W


def _tab_transpose_body(t0_ref, t1_ref, t2_ref, t3_ref, out_ref):
    for r, ref in enumerate((t0_ref, t1_ref, t2_ref, t3_ref)):
        x = ref[...]
        out_ref[:, r * EMB:(r + 1) * EMB] = jnp.transpose(x, (1, 0)) * SCALE


def _transpose_table(table_t):
    def _imap(r, g):
        return (0, jnp.minimum(4 * g + r, _LAST_BLK))

    specs = [
        pl.BlockSpec((EMB, W), functools.partial(_imap, r)) for r in range(4)
    ]
    return pl.pallas_call(
        _tab_transpose_body,
        grid=(_NBLK,),
        in_specs=specs,
        out_specs=pl.BlockSpec((W, 4 * EMB), lambda g: (g, 0)),
        out_shape=jax.ShapeDtypeStruct((PROWS, 4 * EMB), jnp.float32),
    )(table_t, table_t, table_t, table_t)


def kernel(tokens, table):
    v = tokens.T.astype(jnp.int32)
    _LW = W.bit_length() - 1
    tok_t = 4 * (((v >> (_LW + 2)) << _LW) + (v & (W - 1))) + ((v >> _LW) & 3)
    tab_rm = _transpose_table(table.T)
    tab2 = tab_rm.reshape(4 * PROWS, EMB)
    out5 = _emb_kernel(tok_t, tab2)
    out = jnp.transpose(out5, (2, 4, 0, 1, 3))
    return out.reshape(NTOK, SEQ, EMB)


# speedup vs baseline: 1.8237x; 1.7634x over previous
import math
import functools

import jax
import jax.numpy as jnp
from jax import lax
from jax.experimental import pallas as pl
from jax.experimental.pallas import tpu as pltpu
from jax.experimental.pallas import tpu_sc as plsc

VOCAB = 1000000
EMB = 32
SCALE = math.sqrt(float(EMB))

NC = 2
NS = 16
NW = NC * NS

SEQ = 200
NTOK = 4096
IW = NTOK // NW
K = 5
NCHUNK = SEQ // K
G = NCHUNK // 2
TPAD = 136

_mesh = plsc.VectorSubcoreMesh(core_axis_name="c", subcore_axis_name="s")


@functools.partial(
    pl.kernel,
    out_type=jax.ShapeDtypeStruct((SEQ, EMB // 8, NW, 8, IW), jnp.float32),
    mesh=_mesh,
    scratch_types=[
        pltpu.VMEM((K, IW), jnp.int32),
        pltpu.VMEM((K, IW), jnp.int32),
        pltpu.VMEM((K * IW, EMB), jnp.float32),
        pltpu.VMEM((K * IW, EMB), jnp.float32),
        pltpu.VMEM((K, EMB // 8, 8, TPAD), jnp.float32),
        pltpu.VMEM((K, EMB // 8, 8, TPAD), jnp.float32),
        pltpu.SemaphoreType.DMA,
        pltpu.SemaphoreType.DMA,
        pltpu.SemaphoreType.DMA,
        pltpu.SemaphoreType.DMA,
    ],
    compiler_params=pltpu.CompilerParams(
        use_tc_tiling_on_sc=False, needs_layout_passes=False
    ),
)
def _emb_kernel(tok_hbm, tab_hbm, out_hbm, idx0, idx1, rows0, rows1,
                tb0, tb1, sg0, sg1, sw0, sw1):
    wid = lax.axis_index("s") * NC + lax.axis_index("c")
    i0 = wid * IW
    idx = (idx0, idx1)
    rows = (rows0, rows1)
    tb = (tb0, tb1)
    sg = (sg0, sg1)
    sw = (sw0, sw1)

    ev = lax.iota(jnp.int32, 16)
    re_v = ev & 7

    def fire(ci, b):
        rb = ci * K
        pltpu.sync_copy(tok_hbm.at[pl.ds(rb, K), pl.ds(i0, IW)], idx[b])
        for j in range(K):
            pltpu.async_copy(
                tab_hbm.at[idx[b].at[j]],
                rows[b].at[pl.ds(j * IW, IW)],
                sg[b],
            )

    def drain_gather(b):
        pltpu.make_async_copy(
            tab_hbm.at[pl.ds(0, K * IW)],
            rows[b],
            sg[b],
        ).wait()

    def transpose_scale(b):
        for kk in range(K):
            kvec = jnp.full((16,), kk, dtype=jnp.int32)

            @plsc.parallel_loop(0, IW, step=1, unroll=4)
            def _(i):
                ivec = jnp.full((16,), i, dtype=jnp.int32)
                x0 = rows[b][kk * IW + i, pl.ds(0, 16)]
                plsc.store_scatter(tb[b], [kvec, ev >> 3, re_v, ivec], x0)
                x1 = rows[b][kk * IW + i, pl.ds(16, 16)]
                plsc.store_scatter(tb[b], [kvec, 2 + (ev >> 3), re_v, ivec], x1)

    def writeback(ci, b):
        rb = ci * K
        pltpu.async_copy(
            tb[b].at[:, :, :, pl.ds(0, IW)],
            out_hbm.at[pl.ds(rb, K), :, wid, :, :],
            sw[b],
        )

    def drain_wb(b):
        pltpu.make_async_copy(
            tb[b].at[:, :, :, pl.ds(0, IW)],
            out_hbm.at[pl.ds(0, K), :, 0, :, :],
            sw[b],
        ).wait()

    fire(0, 0)

    def outer(g, carry):
        ci0 = 2 * g

        @pl.when(g > 0)
        def _():
            drain_wb(1)

        fire(ci0 + 1, 1)
        drain_gather(0)
        transpose_scale(0)
        writeback(ci0, 0)

        drain_gather(1)
        transpose_scale(1)

        @pl.when(g < G - 1)
        def _():
            drain_wb(0)
            fire(ci0 + 2, 0)

        writeback(ci0 + 1, 1)
        return carry

    lax.fori_loop(0, G, outer, 0)
    drain_wb(0)
    drain_wb(1)


W = 8192
_NBLK = -(-VOCAB // (4 * W))
PROWS = _NBLK * W
_LAST_BLK = (VOCAB - 1) // W


def _tab_transpose_body(t0_ref, t1_ref, t2_ref, t3_ref, out_ref):
    x = jnp.concatenate(
        [t0_ref[...], t1_ref[...], t2_ref[...], t3_ref[...]], axis=0
    )
    out_ref[...] = jnp.transpose(x, (1, 0)) * SCALE


def _transpose_table(table_t):
    def _imap(r, g):
        return (0, jnp.minimum(4 * g + r, _LAST_BLK))

    specs = [
        pl.BlockSpec((EMB, W), functools.partial(_imap, r)) for r in range(4)
    ]
    return pl.pallas_call(
        _tab_transpose_body,
        grid=(_NBLK,),
        in_specs=specs,
        out_specs=pl.BlockSpec((W, 4 * EMB), lambda g: (g, 0)),
        out_shape=jax.ShapeDtypeStruct((PROWS, 4 * EMB), jnp.float32),
    )(table_t, table_t, table_t, table_t)


def kernel(tokens, table):
    v = tokens.T.astype(jnp.int32)
    _LW = W.bit_length() - 1
    tok_t = 4 * (((v >> (_LW + 2)) << _LW) + (v & (W - 1))) + ((v >> _LW) & 3)
    tab_rm = _transpose_table(table.T)
    tab2 = tab_rm.reshape(4 * PROWS, EMB)
    out5 = _emb_kernel(tok_t, tab2)
    out = jnp.transpose(out5, (2, 4, 0, 1, 3))
    return out.reshape(NTOK, SEQ, EMB)


# speedup vs baseline: 1.8372x; 1.0074x over previous
import math
import functools

import jax
import jax.numpy as jnp
from jax import lax
from jax.experimental import pallas as pl
from jax.experimental.pallas import tpu as pltpu
from jax.experimental.pallas import tpu_sc as plsc

VOCAB = 1000000
EMB = 32
SCALE = math.sqrt(float(EMB))

NC = 2
NS = 16
NW = NC * NS

SEQ = 200
NTOK = 4096
IW = NTOK // NW
K = 5
NCHUNK = SEQ // K
G = NCHUNK // 2
TPAD = 136

_mesh = plsc.VectorSubcoreMesh(core_axis_name="c", subcore_axis_name="s")


@functools.partial(
    pl.kernel,
    out_type=jax.ShapeDtypeStruct((SEQ, EMB // 8, NW, 8, IW), jnp.float32),
    mesh=_mesh,
    scratch_types=[
        pltpu.VMEM((K, IW), jnp.int32),
        pltpu.VMEM((K, IW), jnp.int32),
        pltpu.VMEM((K * IW, EMB), jnp.float32),
        pltpu.VMEM((K * IW, EMB), jnp.float32),
        pltpu.VMEM((K, EMB // 8, 8, TPAD), jnp.float32),
        pltpu.VMEM((K, EMB // 8, 8, TPAD), jnp.float32),
        pltpu.SemaphoreType.DMA,
        pltpu.SemaphoreType.DMA,
        pltpu.SemaphoreType.DMA,
        pltpu.SemaphoreType.DMA,
    ],
    compiler_params=pltpu.CompilerParams(
        use_tc_tiling_on_sc=False, needs_layout_passes=False
    ),
)
def _emb_kernel(tok_hbm, tab_hbm, out_hbm, idx0, idx1, rows0, rows1,
                tb0, tb1, sg0, sg1, sw0, sw1):
    wid = lax.axis_index("s") * NC + lax.axis_index("c")
    i0 = wid * IW
    idx = (idx0, idx1)
    rows = (rows0, rows1)
    tb = (tb0, tb1)
    sg = (sg0, sg1)
    sw = (sw0, sw1)

    ev = lax.iota(jnp.int32, 16)
    re_v = ev & 7

    def fire(ci, b):
        rb = ci * K
        pltpu.sync_copy(tok_hbm.at[pl.ds(rb, K), pl.ds(i0, IW)], idx[b])
        for j in range(K):
            pltpu.async_copy(
                tab_hbm.at[idx[b].at[j]],
                rows[b].at[pl.ds(j * IW, IW)],
                sg[b],
            )

    def drain_gather(b):
        pltpu.make_async_copy(
            tab_hbm.at[pl.ds(0, K * IW)],
            rows[b],
            sg[b],
        ).wait()

    def transpose_scale(b):
        for kk in range(K):
            kvec = jnp.full((16,), kk, dtype=jnp.int32)

            @plsc.parallel_loop(0, IW, step=1, unroll=4)
            def _(i):
                ivec = jnp.full((16,), i, dtype=jnp.int32)
                x0 = rows[b][kk * IW + i, pl.ds(0, 16)]
                plsc.store_scatter(tb[b], [kvec, ev >> 3, re_v, ivec], x0)
                x1 = rows[b][kk * IW + i, pl.ds(16, 16)]
                plsc.store_scatter(tb[b], [kvec, 2 + (ev >> 3), re_v, ivec], x1)

    def writeback(ci, b):
        rb = ci * K
        pltpu.async_copy(
            tb[b].at[:, :, :, pl.ds(0, IW)],
            out_hbm.at[pl.ds(rb, K), :, wid, :, :],
            sw[b],
        )

    def drain_wb(b):
        pltpu.make_async_copy(
            tb[b].at[:, :, :, pl.ds(0, IW)],
            out_hbm.at[pl.ds(0, K), :, 0, :, :],
            sw[b],
        ).wait()

    fire(0, 0)

    def outer(g, carry):
        ci0 = 2 * g

        @pl.when(g > 0)
        def _():
            drain_wb(1)

        fire(ci0 + 1, 1)
        drain_gather(0)
        transpose_scale(0)
        writeback(ci0, 0)

        drain_gather(1)
        transpose_scale(1)

        @pl.when(g < G - 1)
        def _():
            drain_wb(0)
            fire(ci0 + 2, 0)

        writeback(ci0 + 1, 1)
        return carry

    lax.fori_loop(0, G, outer, 0)
    drain_wb(0)
    drain_wb(1)


W = 16384
_NBLK = -(-VOCAB // (4 * W))
PROWS = _NBLK * W
_LAST_BLK = (VOCAB - 1) // W


def _tab_transpose_body(t0_ref, t1_ref, t2_ref, t3_ref, out_ref):
    x = jnp.concatenate(
        [t0_ref[...], t1_ref[...], t2_ref[...], t3_ref[...]], axis=0
    )
    out_ref[...] = jnp.transpose(x, (1, 0)) * SCALE


def _transpose_table(table_t):
    def _imap(r, g):
        return (0, jnp.minimum(4 * g + r, _LAST_BLK))

    specs = [
        pl.BlockSpec((EMB, W), functools.partial(_imap, r)) for r in range(4)
    ]
    return pl.pallas_call(
        _tab_transpose_body,
        grid=(_NBLK,),
        in_specs=specs,
        out_specs=pl.BlockSpec((W, 4 * EMB), lambda g: (g, 0)),
        out_shape=jax.ShapeDtypeStruct((PROWS, 4 * EMB), jnp.float32),
    )(table_t, table_t, table_t, table_t)


def kernel(tokens, table):
    v = tokens.T.astype(jnp.int32)
    _LW = W.bit_length() - 1
    tok_t = 4 * (((v >> (_LW + 2)) << _LW) + (v & (W - 1))) + ((v >> _LW) & 3)
    tab_rm = _transpose_table(table.T)
    tab2 = tab_rm.reshape(4 * PROWS, EMB)
    out5 = _emb_kernel(tok_t, tab2)
    out = jnp.transpose(out5, (2, 4, 0, 1, 3))
    return out.reshape(NTOK, SEQ, EMB)
